# Initial kernel scaffold; baseline (speedup 1.0000x reference)
import jax
import jax.numpy as jnp
from jax.experimental import pallas as pl


def kernel(input_tensor):
    raise NotImplementedError("write your pallas kernel here")


# speedup vs baseline: 1.3429x; 1.3429x over previous
import functools

import jax
import jax.numpy as jnp
from jax import lax
from jax.experimental import pallas as pl
from jax.experimental.pallas import tpu as pltpu
from jax.experimental.pallas import tpu_sc as plsc

B, H, W, C = 8, 224, 224, 96
N = H * W * C
NT = B * N
NC, NS = 2, 16
NW = NC * NS
QW = N // 4
S = 24576
CHUNKS = QW // S

_mesh = plsc.VectorSubcoreMesh(core_axis_name="c", subcore_axis_name="s")


@functools.partial(
    pl.kernel,
    mesh=_mesh,
    out_type=jax.ShapeDtypeStruct((NT,), jnp.float32),
    scratch_types=[
        pltpu.VMEM((S,), jnp.float32),
        pltpu.VMEM((S // 2,), jnp.float32),
        pltpu.VMEM((S // 2,), jnp.float32),
    ],
    compiler_params=pltpu.CompilerParams(needs_layout_passes=False),
)
def _deinterleave(in_hbm, out_hbm, inb, eb, ob):
    wid = lax.axis_index("s") * NC + lax.axis_index("c")
    ex = wid // 4
    q = wid % 4
    in_base = ex * N + q * QW
    oe_base = ex * N + q * (QW // 2)
    oo_base = ex * N + N // 2 + q * (QW // 2)
    lane2 = lax.iota(jnp.int32, 16) * 2

    def chunk_body(k, _):
        off = k * S
        src = pl.multiple_of(in_base + off, 8)
        pltpu.sync_copy(in_hbm.at[pl.ds(src, S)], inb)

        def shuf(i, _):
            idx = i * 32 + lane2
            eb[pl.ds(i * 16, 16)] = plsc.load_gather(inb, [idx])
            ob[pl.ds(i * 16, 16)] = plsc.load_gather(inb, [idx + 1])
            return 0

        lax.fori_loop(0, S // 32, shuf, 0)
        dste = pl.multiple_of(oe_base + off // 2, 8)
        dsto = pl.multiple_of(oo_base + off // 2, 8)
        pltpu.sync_copy(eb, out_hbm.at[pl.ds(dste, S // 2)])
        pltpu.sync_copy(ob, out_hbm.at[pl.ds(dsto, S // 2)])
        return 0

    lax.fori_loop(0, CHUNKS, chunk_body, 0)


def kernel(input_tensor):
    flat = input_tensor.reshape(-1)
    out = _deinterleave(flat)
    return out.reshape(B, 2, N // 2)


# speedup vs baseline: 1.3868x; 1.0327x over previous
import functools

import jax
import jax.numpy as jnp
from jax import lax
from jax.experimental import pallas as pl
from jax.experimental.pallas import tpu as pltpu
from jax.experimental.pallas import tpu_sc as plsc

B, H, W, C = 8, 224, 224, 96
N = H * W * C
NT = B * N
NC, NS = 2, 16
NW = NC * NS
QW = N // 4
S = 28672
CHUNKS = QW // S
NBUF = 2

_mesh = plsc.VectorSubcoreMesh(core_axis_name="c", subcore_axis_name="s")


@functools.partial(
    pl.kernel,
    mesh=_mesh,
    out_type=jax.ShapeDtypeStruct((NT,), jnp.float32),
    scratch_types=[
        pltpu.VMEM((NBUF, S), jnp.float32),
        pltpu.VMEM((NBUF, S // 2), jnp.float32),
        pltpu.VMEM((NBUF, S // 2), jnp.float32),
        pltpu.SemaphoreType.DMA,
        pltpu.SemaphoreType.DMA,
        pltpu.SemaphoreType.DMA,
        pltpu.SemaphoreType.DMA,
    ],
    compiler_params=pltpu.CompilerParams(
        needs_layout_passes=False, use_tc_tiling_on_sc=False
    ),
)
def _deinterleave(in_hbm, out_hbm, inb, eb, ob, sin0, sin1, sout0, sout1):
    wid = lax.axis_index("s") * NC + lax.axis_index("c")
    ex = wid // 4
    q = wid % 4
    in_base = ex * N + q * QW
    oe_base = ex * N + q * (QW // 2)
    oo_base = ex * N + N // 2 + q * (QW // 2)
    lane2 = lax.iota(jnp.int32, 16) * 2
    sins = (sin0, sin1)
    souts = (sout0, sout1)

    def in_copy(k, slot):
        src = pl.multiple_of(in_base + k * S, 8)
        return pltpu.make_async_copy(
            in_hbm.at[pl.ds(src, S)], inb.at[slot], sins[slot]
        )

    def out_copies(k, slot):
        dste = pl.multiple_of(oe_base + k * (S // 2), 8)
        dsto = pl.multiple_of(oo_base + k * (S // 2), 8)
        ce = pltpu.make_async_copy(
            eb.at[slot], out_hbm.at[pl.ds(dste, S // 2)], souts[slot]
        )
        co = pltpu.make_async_copy(
            ob.at[slot], out_hbm.at[pl.ds(dsto, S // 2)], souts[slot]
        )
        return ce, co

    in_copy(0, 0).start()
    in_copy(1, 1).start()

    def outer(k0, _):
        for slot in range(NBUF):
            k = k0 * NBUF + slot
            in_copy(k, slot).wait()

            @pl.when(k0 > 0)
            def _():
                ce, co = out_copies(k, slot)
                ce.wait()
                co.wait()

            def shuf(i, _):
                idx = i * 32 + lane2
                src = inb.at[slot]
                eb[slot, pl.ds(i * 16, 16)] = plsc.load_gather(src, [idx])
                ob[slot, pl.ds(i * 16, 16)] = plsc.load_gather(src, [idx + 1])
                return 0

            lax.fori_loop(0, S // 32, shuf, 0)
            ce, co = out_copies(k, slot)
            ce.start()
            co.start()

            @pl.when(k + NBUF < CHUNKS)
            def _():
                in_copy(k + NBUF, slot).start()

        return 0

    lax.fori_loop(0, CHUNKS // NBUF, outer, 0)
    for slot in range(NBUF):
        ce, co = out_copies(CHUNKS - NBUF + slot, slot)
        ce.wait()
        co.wait()


def kernel(input_tensor):
    flat = input_tensor.reshape(-1)
    out = _deinterleave(flat)
    return out.reshape(B, 2, N // 2)


# speedup vs baseline: 7.0723x; 5.0999x over previous
import functools

import jax
import jax.numpy as jnp
from jax import lax
from jax.experimental import pallas as pl
from jax.experimental.pallas import tpu as pltpu
from jax.experimental.pallas import tpu_sc as plsc

B, H, W, C = 8, 224, 224, 96
N = H * W * C
ND2 = N // 2
SLAB_OUT = W * C // 2
NC, NS = 2, 16
HPW = H // 4
NBUF = 2

_mesh = plsc.VectorSubcoreMesh(core_axis_name="c", subcore_axis_name="s")


@functools.partial(
    pl.kernel,
    mesh=_mesh,
    out_type=jax.ShapeDtypeStruct((B, 2, ND2), jnp.float32),
    scratch_types=[
        pltpu.VMEM((NBUF, C, W), jnp.float32),
        pltpu.VMEM((NBUF, 2, SLAB_OUT), jnp.float32),
        pltpu.SemaphoreType.DMA,
        pltpu.SemaphoreType.DMA,
        pltpu.SemaphoreType.DMA,
        pltpu.SemaphoreType.DMA,
    ],
    compiler_params=pltpu.CompilerParams(
        needs_layout_passes=False, use_tc_tiling_on_sc=True
    ),
)
def _deinterleave(in_hbm, out_hbm, ibuf, obuf, sin0, sin1, sout0, sout1):
    wid = lax.axis_index("s") * NC + lax.axis_index("c")
    ex = wid // 4
    q = wid % 4
    h0 = q * HPW
    sins = (sin0, sin1)
    souts = (sout0, sout1)
    lane = lax.iota(jnp.int32, 16)

    rowcol = []
    for g3 in range(3):
        for r in range(2):
            for i8 in range(8):
                vb = lane + (128 * g3 + 16 * i8)
                wv = vb // 48
                rowv = 2 * (vb - 48 * wv) + r
                rowcol.append((g3, r, i8, rowv, wv))

    def in_copy(h, slot):
        return pltpu.make_async_copy(
            in_hbm.at[ex, h0 + h], ibuf.at[slot], sins[slot]
        )

    def out_copy(h, slot):
        dst = pl.multiple_of((h0 + h) * SLAB_OUT, 128)
        return pltpu.make_async_copy(
            obuf.at[slot], out_hbm.at[ex, :, pl.ds(dst, SLAB_OUT)], souts[slot]
        )

    in_copy(0, 0).start()
    in_copy(1, 1).start()

    def outer(k0, _):
        for slot in range(NBUF):
            h = k0 * NBUF + slot
            in_copy(h, slot).wait()

            @pl.when(k0 > 0)
            def _():
                out_copy(h, slot).wait()

            src = ibuf.at[slot]

            def body(a, _):
                for g3, r, i8, rowv, wv in rowcol:
                    colv = wv + 8 * a
                    val = plsc.load_gather(src, [rowv, colv])
                    obuf[slot, r, pl.ds((3 * a + g3) * 128 + 16 * i8, 16)] = val
                return 0

            lax.fori_loop(0, W // 8, body, 0)
            out_copy(h, slot).start()

            @pl.when(h + NBUF < HPW)
            def _():
                in_copy(h + NBUF, slot).start()

        return 0

    lax.fori_loop(0, HPW // NBUF, outer, 0)
    for slot in range(NBUF):
        out_copy(HPW - NBUF + slot, slot).wait()


def kernel(input_tensor):
    xt = jnp.transpose(input_tensor, (0, 1, 3, 2))
    return _deinterleave(xt)


# speedup vs baseline: 7.0740x; 1.0002x over previous
import functools

import jax
import jax.numpy as jnp
from jax import lax
from jax.experimental import pallas as pl
from jax.experimental.pallas import tpu as pltpu
from jax.experimental.pallas import tpu_sc as plsc

B, H, W, C = 8, 224, 224, 96
N = H * W * C
ND2 = N // 2
SLAB_OUT = W * C // 2
NC, NS = 2, 16
HPW = H // 4
NBUF = 2

_mesh = plsc.VectorSubcoreMesh(core_axis_name="c", subcore_axis_name="s")


@functools.partial(
    pl.kernel,
    mesh=_mesh,
    out_type=jax.ShapeDtypeStruct((B, 2, ND2), jnp.float32),
    scratch_types=[
        pltpu.VMEM((NBUF, C, W), jnp.float32),
        pltpu.VMEM((NBUF, 2, SLAB_OUT), jnp.float32),
        pltpu.SemaphoreType.DMA,
        pltpu.SemaphoreType.DMA,
        pltpu.SemaphoreType.DMA,
        pltpu.SemaphoreType.DMA,
    ],
    compiler_params=pltpu.CompilerParams(
        needs_layout_passes=False,
        use_tc_tiling_on_sc=True,
        disable_bounds_checks=True,
    ),
)
def _deinterleave(in_hbm, out_hbm, ibuf, obuf, sin0, sin1, sout0, sout1):
    wid = lax.axis_index("s") * NC + lax.axis_index("c")
    ex = wid // 4
    q = wid % 4
    h0 = q * HPW
    sins = (sin0, sin1)
    souts = (sout0, sout1)
    lane = lax.iota(jnp.int32, 16)

    rowcol = []
    for g3 in range(3):
        for r in range(2):
            for i8 in range(8):
                vb = lane + (128 * g3 + 16 * i8)
                wv = vb // 48
                rowv = 2 * (vb - 48 * wv) + r
                rowcol.append((g3, r, i8, rowv, wv))

    def in_copy(h, slot):
        return pltpu.make_async_copy(
            in_hbm.at[ex, h0 + h], ibuf.at[slot], sins[slot]
        )

    def out_copy(h, slot):
        dst = pl.multiple_of((h0 + h) * SLAB_OUT, 128)
        return pltpu.make_async_copy(
            obuf.at[slot], out_hbm.at[ex, :, pl.ds(dst, SLAB_OUT)], souts[slot]
        )

    in_copy(0, 0).start()
    in_copy(1, 1).start()

    def outer(k0, _):
        for slot in range(NBUF):
            h = k0 * NBUF + slot
            in_copy(h, slot).wait()

            @pl.when(k0 > 0)
            def _():
                out_copy(h, slot).wait()

            src = ibuf.at[slot]

            def body(a, _):
                for g3, r, i8, rowv, wv in rowcol:
                    colv = wv + 8 * a
                    val = plsc.load_gather(src, [rowv, colv])
                    obuf[slot, r, pl.ds((3 * a + g3) * 128 + 16 * i8, 16)] = val
                return 0

            lax.fori_loop(0, W // 8, body, 0)
            out_copy(h, slot).start()

            @pl.when(h + NBUF < HPW)
            def _():
                in_copy(h + NBUF, slot).start()

        return 0

    lax.fori_loop(0, HPW // NBUF, outer, 0)
    for slot in range(NBUF):
        out_copy(HPW - NBUF + slot, slot).wait()


def kernel(input_tensor):
    xt = jnp.transpose(input_tensor, (0, 1, 3, 2))
    return _deinterleave(xt)


# speedup vs baseline: 20.2636x; 2.8645x over previous
import functools

import jax
import jax.numpy as jnp
from jax import lax
from jax.experimental import pallas as pl
from jax.experimental.pallas import tpu as pltpu
from jax.experimental.pallas import tpu_sc as plsc

B, H, W, C = 8, 224, 224, 96
N = H * W * C
ND2 = N // 2
SLAB_OUT = W * C // 2
NC, NS = 2, 16
HPW = H // 4
NBUF = 2

_mesh = plsc.VectorSubcoreMesh(core_axis_name="c", subcore_axis_name="s")


@functools.partial(
    pl.kernel,
    mesh=_mesh,
    out_type=jax.ShapeDtypeStruct((B, 2, ND2), jnp.float32),
    scratch_types=[
        pltpu.VMEM((NBUF, C, W), jnp.float32),
        pltpu.VMEM((NBUF, 2, SLAB_OUT), jnp.float32),
        pltpu.SemaphoreType.DMA,
        pltpu.SemaphoreType.DMA,
        pltpu.SemaphoreType.DMA,
        pltpu.SemaphoreType.DMA,
    ],
    compiler_params=pltpu.CompilerParams(
        needs_layout_passes=False,
        use_tc_tiling_on_sc=True,
        disable_bounds_checks=True,
    ),
)
def _deinterleave(in_hbm, out_hbm, ibuf, obuf, sin0, sin1, sout0, sout1):
    wid = lax.axis_index("s") * NC + lax.axis_index("c")
    ex = wid // 4
    q = wid % 4
    h0 = q * HPW
    sins = (sin0, sin1)
    souts = (sout0, sout1)
    lane = lax.iota(jnp.int32, 16)

    lane48 = lane * 48
    colvs = [lane + 16 * wb for wb in range(W // 16)]

    def in_copy(h, slot):
        return pltpu.make_async_copy(
            in_hbm.at[ex, h0 + h], ibuf.at[slot], sins[slot]
        )

    def out_copy(h, slot):
        dst = pl.multiple_of((h0 + h) * SLAB_OUT, 128)
        return pltpu.make_async_copy(
            obuf.at[slot], out_hbm.at[ex, :, pl.ds(dst, SLAB_OUT)], souts[slot]
        )

    in_copy(0, 0).start()
    in_copy(1, 1).start()

    def outer(k0, _):
        for slot in range(NBUF):
            h = k0 * NBUF + slot
            in_copy(h, slot).wait()

            @pl.when(k0 > 0)
            def _():
                out_copy(h, slot).wait()

            src = ibuf.at[slot]

            def body(d, _):
                t = d + lane
                c2 = jnp.where(t >= 48, t - 48, t)
                jbase = lane48 + c2
                dst = obuf.at[slot]
                for r in range(2):
                    rowv = c2 * 2 + r
                    ridx = lane * 0 + r
                    for wb in range(W // 16):
                        val = plsc.load_gather(src, [rowv, colvs[wb]])
                        jv = jbase + 768 * wb
                        plsc.store_scatter(dst, [ridx, jv], val)
                return 0

            lax.fori_loop(0, 48, body, 0)
            out_copy(h, slot).start()

            @pl.when(h + NBUF < HPW)
            def _():
                in_copy(h + NBUF, slot).start()

        return 0

    lax.fori_loop(0, HPW // NBUF, outer, 0)
    for slot in range(NBUF):
        out_copy(HPW - NBUF + slot, slot).wait()


def kernel(input_tensor):
    xt = jnp.transpose(input_tensor, (0, 1, 3, 2))
    return _deinterleave(xt)


# speedup vs baseline: 42.8600x; 2.1151x over previous
import functools

import jax
import jax.numpy as jnp
from jax import lax
from jax.experimental import pallas as pl
from jax.experimental.pallas import tpu as pltpu
from jax.experimental.pallas import tpu_sc as plsc

B, H, W, C = 8, 224, 224, 96
N = H * W * C
ND2 = N // 2
SLAB_OUT = W * C // 2
NC, NS = 2, 16
HPW = H // 4
NBUF = 2

_mesh = plsc.VectorSubcoreMesh(core_axis_name="c", subcore_axis_name="s")


@functools.partial(
    pl.kernel,
    mesh=_mesh,
    out_type=jax.ShapeDtypeStruct((B, 2, ND2), jnp.float32),
    scratch_types=[
        pltpu.VMEM((NBUF, C, W), jnp.float32),
        pltpu.VMEM((NBUF, 2, SLAB_OUT), jnp.float32),
        pltpu.SemaphoreType.DMA,
        pltpu.SemaphoreType.DMA,
        pltpu.SemaphoreType.DMA,
        pltpu.SemaphoreType.DMA,
    ],
    compiler_params=pltpu.CompilerParams(
        needs_layout_passes=False,
        use_tc_tiling_on_sc=True,
        disable_bounds_checks=True,
    ),
)
def _deinterleave(in_hbm, out_hbm, ibuf, obuf, sin0, sin1, sout0, sout1):
    wid = lax.axis_index("s") * NC + lax.axis_index("c")
    ex = wid // 4
    q = wid % 4
    h0 = q * HPW
    sins = (sin0, sin1)
    souts = (sout0, sout1)
    lane = lax.iota(jnp.int32, 16)

    lane48 = lane * 48
    colvs = [lane + 16 * wb for wb in range(W // 16)]
    ridxs = [lane * 0, lane * 0 + 1]

    def in_copy(h, slot):
        return pltpu.make_async_copy(
            in_hbm.at[ex, h0 + h], ibuf.at[slot], sins[slot]
        )

    def out_copy(h, slot):
        dst = pl.multiple_of((h0 + h) * SLAB_OUT, 128)
        return pltpu.make_async_copy(
            obuf.at[slot], out_hbm.at[ex, :, pl.ds(dst, SLAB_OUT)], souts[slot]
        )

    in_copy(0, 0).start()
    in_copy(1, 1).start()

    def outer(k0, _):
        for slot in range(NBUF):
            h = k0 * NBUF + slot
            in_copy(h, slot).wait()

            @pl.when(k0 > 0)
            def _():
                out_copy(h, slot).wait()

            src = ibuf.at[slot]

            dst = obuf.at[slot]

            @plsc.parallel_loop(0, 48)
            def _loop(d):
                t = d + lane
                c2 = jnp.where(t >= 48, t - 48, t)
                jbase = lane48 + c2
                c22 = c2 * 2
                for r in range(2):
                    rowv = c22 + r if r else c22
                    for wb in range(W // 16):
                        val = plsc.load_gather(src, [rowv, colvs[wb]])
                        jv = jbase + 768 * wb
                        plsc.store_scatter(dst, [ridxs[r], jv], val)
            out_copy(h, slot).start()

            @pl.when(h + NBUF < HPW)
            def _():
                in_copy(h + NBUF, slot).start()

        return 0

    lax.fori_loop(0, HPW // NBUF, outer, 0)
    for slot in range(NBUF):
        out_copy(HPW - NBUF + slot, slot).wait()


def kernel(input_tensor):
    xt = jnp.transpose(input_tensor, (0, 1, 3, 2))
    return _deinterleave(xt)
